# baseline (speedup 1.0000x reference)
import functools

import jax
import jax.numpy as jnp
from jax import lax
from jax.experimental import pallas as pl
from jax.experimental.pallas import tpu as pltpu
from jax.experimental.pallas import tpu_sc as plsc

EMBED_DIM = 64
HIDDEN_DIM = 128
OUT_DIM = 32
BATCH = 16384
HIST = 200
VOCAB = 1000000

NUM_WORKERS = 32
E_PER_W = BATCH // NUM_WORKERS
INV_H = 1.0 / HIST

CB = 256
NBLK_FULL = VOCAB // CB
TAIL_C0 = NBLK_FULL * CB
TAIL_W = VOCAB - TAIL_C0

CHUNK = 2
ROWS = CHUNK * HIST
NCHUNK = E_PER_W // CHUNK
XBLK = 32
NRING = 4

_mesh = plsc.VectorSubcoreMesh(core_axis_name="c", subcore_axis_name="s")


@functools.partial(
    pl.kernel,
    mesh=_mesh,
    out_type=jax.ShapeDtypeStruct((VOCAB // 2, 2 * EMBED_DIM), jnp.float32),
    scratch_types=[
        pltpu.VMEM((EMBED_DIM, CB), jnp.float32),
        pltpu.VMEM((EMBED_DIM, CB), jnp.float32),
        pltpu.VMEM((CB // 2, 2 * EMBED_DIM), jnp.float32),
        pltpu.VMEM((CB // 2, 2 * EMBED_DIM), jnp.float32),
        pltpu.VMEM((EMBED_DIM, TAIL_W), jnp.float32),
        pltpu.VMEM((TAIL_W // 2, 2 * EMBED_DIM), jnp.float32),
        pltpu.SemaphoreType.DMA,
        pltpu.SemaphoreType.DMA,
        pltpu.SemaphoreType.DMA,
        pltpu.SemaphoreType.DMA,
    ],
    compiler_params=pltpu.CompilerParams(needs_layout_passes=False),
)
def _relin(tabT_hbm, t2_hbm, vi0, vi1, vo0, vo1, vit, vot,
           si0, si1, so0, so1):
    wid = lax.axis_index("s") * 2 + lax.axis_index("c")
    vis = (vi0, vi1)
    vos = (vo0, vo1)
    sis = (si0, si1)
    sos = (so0, so1)
    row_iotas = [lax.iota(jnp.int32, 16) + 16 * q for q in range(4)]

    def blk(k):
        return wid + NUM_WORKERS * k

    def start_in(k, j):
        pltpu.make_async_copy(
            tabT_hbm.at[:, pl.ds(blk(k) * CB, CB)], vis[j], sis[j]).start()

    def wait_in(j):
        pltpu.make_async_copy(
            tabT_hbm.at[:, pl.ds(0, CB)], vis[j], sis[j]).wait()

    def start_out(k, j):
        pltpu.make_async_copy(
            vos[j], t2_hbm.at[pl.ds(blk(k) * (CB // 2), CB // 2)],
            sos[j]).start()

    def wait_out(j):
        pltpu.make_async_copy(
            vos[j], t2_hbm.at[pl.ds(0, CB // 2)], sos[j]).wait()

    def transpose(vi, vo, nrows):
        def body(j2, carry, vi=vi, vo=vo):
            for h in range(2):
                col = jnp.full((16,), 2 * j2 + h, jnp.int32)
                for q in range(4):
                    vals = plsc.load_gather(vi, [row_iotas[q], col])
                    vo[j2, pl.ds(64 * h + 16 * q, 16)] = vals
            return carry

        lax.fori_loop(0, nrows, body, 0, unroll=2)

    nblk = jnp.where(wid < NBLK_FULL - (NBLK_FULL // NUM_WORKERS) *
                     NUM_WORKERS, NBLK_FULL // NUM_WORKERS + 1,
                     NBLK_FULL // NUM_WORKERS)

    start_in(0, 0)
    start_in(1, 1)

    def pair(p, carry):
        k0 = 2 * p
        for j in range(2):
            k = k0 + j

            @pl.when(k < nblk)
            def _(k=k, j=j):
                wait_in(j)

                @pl.when(p > 0)
                def _():
                    wait_out(j)

                transpose(vis[j], vos[j], CB // 2)
                start_out(k, j)

                @pl.when(k + 2 < nblk)
                def _():
                    start_in(k + 2, j)

        return carry

    lax.fori_loop(0, (NBLK_FULL // NUM_WORKERS + 2) // 2, pair, 0)
    wait_out(0)
    wait_out(1)

    @pl.when(wid == 2)
    def _():
        pltpu.sync_copy(tabT_hbm.at[:, pl.ds(TAIL_C0, TAIL_W)], vit)
        transpose(vit, vot, TAIL_W // 2)
        pltpu.sync_copy(vot, t2_hbm.at[pl.ds(TAIL_C0 // 2, TAIL_W // 2)])


@functools.partial(
    pl.kernel,
    mesh=_mesh,
    out_type=jax.ShapeDtypeStruct((BATCH, EMBED_DIM), jnp.float32),
    scratch_types=[
        pltpu.VMEM((XBLK // CHUNK, ROWS), jnp.int32),
        pltpu.VMEM((XBLK // CHUNK, ROWS), jnp.int32),
        pltpu.VMEM((ROWS, EMBED_DIM), jnp.float32),
        pltpu.VMEM((ROWS, EMBED_DIM), jnp.float32),
        pltpu.VMEM((ROWS, EMBED_DIM), jnp.float32),
        pltpu.VMEM((ROWS, EMBED_DIM), jnp.float32),
        pltpu.VMEM((8, EMBED_DIM), jnp.float32),
        pltpu.SemaphoreType.DMA,
        pltpu.SemaphoreType.DMA,
        pltpu.SemaphoreType.DMA,
        pltpu.SemaphoreType.DMA,
    ],
    compiler_params=pltpu.CompilerParams(use_tc_tiling_on_sc=False),
)
def _pool(x_hbm, t2_hbm, out_hbm,
          xb0, xb1, rb0, rb1, rb2, rb3, stage, g0, g1, g2, g3):
    wid = lax.axis_index("s") * 2 + lax.axis_index("c")
    row_base = wid * E_PER_W
    zero = jnp.zeros((16,), jnp.float32)
    xbufs = (xb0, xb1)
    rbufs = (rb0, rb1, rb2, rb3)
    sems = (g0, g1, g2, g3)
    chunks_per_xblk = XBLK // CHUNK

    def stage_x(kx, xbuf):
        pltpu.sync_copy(
            x_hbm.at[pl.ds((row_base + kx * XBLK) // CHUNK,
                           XBLK // CHUNK), :], xbuf)

    def start_gather(c, j):
        kx = c // chunks_per_xblk
        lc = c - kx * chunks_per_xblk
        @pl.when(kx % 2 == 0)
        def _():
            pltpu.make_async_copy(
                t2_hbm.at[xbufs[0].at[lc, :]],
                rbufs[j], sems[j]).start()

        @pl.when(kx % 2 == 1)
        def _():
            pltpu.make_async_copy(
                t2_hbm.at[xbufs[1].at[lc, :]],
                rbufs[j], sems[j]).start()

    def accum(c, j):
        pltpu.make_async_copy(
            t2_hbm.at[xbufs[0].at[0, :]],
            rbufs[j], sems[j]).wait()
        rbuf = rbufs[j]
        for e in range(CHUNK):
            base_r = e * HIST

            def body(i, accs, base_r=base_r, rbuf=rbuf):
                a0, a1, a2, a3 = accs
                a0 = a0 + rbuf[base_r + i, pl.ds(0, 16)]
                a1 = a1 + rbuf[base_r + i, pl.ds(16, 16)]
                a2 = a2 + rbuf[base_r + i, pl.ds(32, 16)]
                a3 = a3 + rbuf[base_r + i, pl.ds(48, 16)]
                return (a0, a1, a2, a3)

            a0, a1, a2, a3 = lax.fori_loop(
                0, HIST, body, (zero, zero, zero, zero), unroll=8)
            srow = (2 * j + e) % 8
            stage[srow, pl.ds(0, 16)] = a0 * INV_H
            stage[srow, pl.ds(16, 16)] = a1 * INV_H
            stage[srow, pl.ds(32, 16)] = a2 * INV_H
            stage[srow, pl.ds(48, 16)] = a3 * INV_H

    stage_x(0, xb0)
    start_gather(0, 0)
    start_gather(1, 1)
    start_gather(2, 2)

    def quad(p, carry):
        for j in range(NRING):
            c = 4 * p + j

            @pl.when(c % chunks_per_xblk == 12)
            def _(c=c):
                kxn = c // chunks_per_xblk + 1

                @pl.when(kxn < E_PER_W // XBLK)
                def _():
                    @pl.when(kxn % 2 == 0)
                    def _():
                        stage_x(kxn, xbufs[0])

                    @pl.when(kxn % 2 == 1)
                    def _():
                        stage_x(kxn, xbufs[1])

            accum(c, j)

            @pl.when(c + 3 < NCHUNK)
            def _(c=c, j=j):
                start_gather(c + 3, (j + 3) % NRING)

        pltpu.sync_copy(
            stage, out_hbm.at[pl.ds(row_base + 8 * p, 8)])
        return carry

    lax.fori_loop(0, NCHUNK // NRING, quad, 0)


def _mlp_body(m_ref, w1_ref, b1_ref, w2_ref, b2_ref, o_ref):
    m = m_ref[...]
    h = lax.dot_general(m, w1_ref[...], (((1,), (0,)), ((), ())),
                        preferred_element_type=jnp.float32)
    h = jnp.maximum(h + b1_ref[...], 0.0)
    o = lax.dot_general(h, w2_ref[...], (((1,), (0,)), ((), ())),
                        preferred_element_type=jnp.float32)
    o = o + b2_ref[...]
    n = jnp.sqrt(jnp.sum(o * o, axis=1, keepdims=True) + 1e-08)
    o_ref[...] = o / n


def _mlp(m, W1, b1, W2, b2):
    blk = 2048
    grid = (BATCH // blk,)
    return pl.pallas_call(
        _mlp_body,
        grid=grid,
        in_specs=[
            pl.BlockSpec((blk, EMBED_DIM), lambda i: (i, 0)),
            pl.BlockSpec((EMBED_DIM, HIDDEN_DIM), lambda i: (0, 0)),
            pl.BlockSpec((1, HIDDEN_DIM), lambda i: (0, 0)),
            pl.BlockSpec((HIDDEN_DIM, OUT_DIM), lambda i: (0, 0)),
            pl.BlockSpec((1, OUT_DIM), lambda i: (0, 0)),
        ],
        out_specs=pl.BlockSpec((blk, OUT_DIM), lambda i: (i, 0)),
        out_shape=jax.ShapeDtypeStruct((BATCH, OUT_DIM), jnp.float32),
    )(m, W1, b1.reshape(1, -1), W2, b2.reshape(1, -1))


def kernel(x, table, W1, b1, W2, b2):
    x = x.astype(jnp.int32)
    t2 = _relin(table.T)
    t2lin = t2.reshape(VOCAB, EMBED_DIM)
    m = _pool(x.reshape(BATCH // CHUNK, ROWS), t2lin)
    return _mlp(m, W1, b1, W2, b2)


# speedup vs baseline: 1.9640x; 1.9640x over previous
import functools

import jax
import jax.numpy as jnp
from jax import lax
from jax.experimental import pallas as pl
from jax.experimental.pallas import tpu as pltpu
from jax.experimental.pallas import tpu_sc as plsc

EMBED_DIM = 64
HIDDEN_DIM = 128
OUT_DIM = 32
BATCH = 16384
HIST = 200
VOCAB = 1000000

NUM_WORKERS = 32
E_PER_W = BATCH // NUM_WORKERS
INV_H = 1.0 / HIST

CB = 256
NBLK_FULL = VOCAB // CB
TAIL_C0 = NBLK_FULL * CB
TAIL_W = VOCAB - TAIL_C0

CHUNK = 2
ROWS = CHUNK * HIST
NCHUNK = E_PER_W // CHUNK
XBLK = 32
NRING = 4

_mesh = plsc.VectorSubcoreMesh(core_axis_name="c", subcore_axis_name="s")


@functools.partial(
    pl.kernel,
    mesh=_mesh,
    out_type=jax.ShapeDtypeStruct((VOCAB // 2, 2 * EMBED_DIM), jnp.float32),
    scratch_types=[
        pltpu.VMEM((EMBED_DIM, CB), jnp.float32),
        pltpu.VMEM((EMBED_DIM, CB), jnp.float32),
        pltpu.VMEM((CB // 2, 2 * EMBED_DIM), jnp.float32),
        pltpu.VMEM((CB // 2, 2 * EMBED_DIM), jnp.float32),
        pltpu.VMEM((EMBED_DIM, TAIL_W), jnp.float32),
        pltpu.VMEM((TAIL_W // 2, 2 * EMBED_DIM), jnp.float32),
        pltpu.SemaphoreType.DMA,
        pltpu.SemaphoreType.DMA,
        pltpu.SemaphoreType.DMA,
        pltpu.SemaphoreType.DMA,
    ],
    compiler_params=pltpu.CompilerParams(needs_layout_passes=False),
)
def _relin(tabT_hbm, t2_hbm, vi0, vi1, vo0, vo1, vit, vot,
           si0, si1, so0, so1):
    wid = lax.axis_index("s") * 2 + lax.axis_index("c")
    vis = (vi0, vi1)
    vos = (vo0, vo1)
    sis = (si0, si1)
    sos = (so0, so1)
    row_iotas = [lax.iota(jnp.int32, 16) + 16 * q for q in range(4)]

    def blk(k):
        return wid + NUM_WORKERS * k

    def start_in(k, j):
        pltpu.make_async_copy(
            tabT_hbm.at[:, pl.ds(blk(k) * CB, CB)], vis[j], sis[j]).start()

    def wait_in(j):
        pltpu.make_async_copy(
            tabT_hbm.at[:, pl.ds(0, CB)], vis[j], sis[j]).wait()

    def start_out(k, j):
        pltpu.make_async_copy(
            vos[j], t2_hbm.at[pl.ds(blk(k) * (CB // 2), CB // 2)],
            sos[j]).start()

    def wait_out(j):
        pltpu.make_async_copy(
            vos[j], t2_hbm.at[pl.ds(0, CB // 2)], sos[j]).wait()

    def transpose(vi, vo, nrows):
        def body(j2, carry, vi=vi, vo=vo):
            for h in range(2):
                col = jnp.full((16,), 2 * j2 + h, jnp.int32)
                for q in range(4):
                    vals = plsc.load_gather(vi, [row_iotas[q], col])
                    vo[j2, pl.ds(64 * h + 16 * q, 16)] = vals
            return carry

        lax.fori_loop(0, nrows, body, 0, unroll=2)

    nblk = jnp.where(wid < NBLK_FULL - (NBLK_FULL // NUM_WORKERS) *
                     NUM_WORKERS, NBLK_FULL // NUM_WORKERS + 1,
                     NBLK_FULL // NUM_WORKERS)

    start_in(0, 0)
    start_in(1, 1)

    def pair(p, carry):
        k0 = 2 * p
        for j in range(2):
            k = k0 + j

            @pl.when(k < nblk)
            def _(k=k, j=j):
                wait_in(j)

                @pl.when(p > 0)
                def _():
                    wait_out(j)

                transpose(vis[j], vos[j], CB // 2)
                start_out(k, j)

                @pl.when(k + 2 < nblk)
                def _():
                    start_in(k + 2, j)

        return carry

    lax.fori_loop(0, (NBLK_FULL // NUM_WORKERS + 2) // 2, pair, 0)
    wait_out(0)
    wait_out(1)

    @pl.when(wid == 2)
    def _():
        pltpu.sync_copy(tabT_hbm.at[:, pl.ds(TAIL_C0, TAIL_W)], vit)
        transpose(vit, vot, TAIL_W // 2)
        pltpu.sync_copy(vot, t2_hbm.at[pl.ds(TAIL_C0 // 2, TAIL_W // 2)])


@functools.partial(
    pl.kernel,
    mesh=_mesh,
    out_type=jax.ShapeDtypeStruct((BATCH, EMBED_DIM), jnp.float32),
    scratch_types=[
        pltpu.VMEM((XBLK // CHUNK, ROWS), jnp.int32),
        pltpu.VMEM((XBLK // CHUNK, ROWS), jnp.int32),
        pltpu.VMEM((ROWS, EMBED_DIM), jnp.float32),
        pltpu.VMEM((ROWS, EMBED_DIM), jnp.float32),
        pltpu.VMEM((ROWS, EMBED_DIM), jnp.float32),
        pltpu.VMEM((ROWS, EMBED_DIM), jnp.float32),
        pltpu.VMEM((8, EMBED_DIM), jnp.float32),
        pltpu.SemaphoreType.DMA,
        pltpu.SemaphoreType.DMA,
        pltpu.SemaphoreType.DMA,
        pltpu.SemaphoreType.DMA,
    ],
    compiler_params=pltpu.CompilerParams(use_tc_tiling_on_sc=False),
)
def _pool(x_hbm, t2_hbm, out_hbm,
          xb0, xb1, rb0, rb1, rb2, rb3, stage, g0, g1, g2, g3):
    wid = lax.axis_index("s") * 2 + lax.axis_index("c")
    row_base = wid * E_PER_W
    zero = jnp.zeros((16,), jnp.float32)
    xbufs = (xb0, xb1)
    rbufs = (rb0, rb1, rb2, rb3)
    sems = (g0, g1, g2, g3)
    chunks_per_xblk = XBLK // CHUNK

    def stage_x(kx, xbuf):
        pltpu.sync_copy(
            x_hbm.at[pl.ds((row_base + kx * XBLK) // CHUNK,
                           XBLK // CHUNK), :], xbuf)

    def start_gather(c, j):
        kx = c // chunks_per_xblk
        lc = c - kx * chunks_per_xblk
        @pl.when(kx % 2 == 0)
        def _():
            pltpu.make_async_copy(
                t2_hbm.at[xbufs[0].at[lc, :]],
                rbufs[j], sems[j]).start()

        @pl.when(kx % 2 == 1)
        def _():
            pltpu.make_async_copy(
                t2_hbm.at[xbufs[1].at[lc, :]],
                rbufs[j], sems[j]).start()

    def accum(c, j):
        pltpu.make_async_copy(
            t2_hbm.at[xbufs[0].at[0, :]],
            rbufs[j], sems[j]).wait()
        rbuf = rbufs[j]
        for e in range(CHUNK):
            base_r = e * HIST

            def body(i, accs, base_r=base_r, rbuf=rbuf):
                a0, a1, a2, a3 = accs
                a0 = a0 + rbuf[base_r + i, pl.ds(0, 16)]
                a1 = a1 + rbuf[base_r + i, pl.ds(16, 16)]
                a2 = a2 + rbuf[base_r + i, pl.ds(32, 16)]
                a3 = a3 + rbuf[base_r + i, pl.ds(48, 16)]
                return (a0, a1, a2, a3)

            a0, a1, a2, a3 = lax.fori_loop(
                0, HIST, body, (zero, zero, zero, zero), unroll=8)
            srow = (2 * j + e) % 8
            stage[srow, pl.ds(0, 16)] = a0 * INV_H
            stage[srow, pl.ds(16, 16)] = a1 * INV_H
            stage[srow, pl.ds(32, 16)] = a2 * INV_H
            stage[srow, pl.ds(48, 16)] = a3 * INV_H

    stage_x(0, xb0)
    start_gather(0, 0)
    start_gather(1, 1)
    start_gather(2, 2)

    def quad(p, carry):
        for j in range(NRING):
            c = 4 * p + j

            @pl.when(c % chunks_per_xblk == 12)
            def _(c=c):
                kxn = c // chunks_per_xblk + 1

                @pl.when(kxn < E_PER_W // XBLK)
                def _():
                    @pl.when(kxn % 2 == 0)
                    def _():
                        stage_x(kxn, xbufs[0])

                    @pl.when(kxn % 2 == 1)
                    def _():
                        stage_x(kxn, xbufs[1])

            accum(c, j)

            @pl.when(c + 3 < NCHUNK)
            def _(c=c, j=j):
                start_gather(c + 3, (j + 3) % NRING)

        pltpu.sync_copy(
            stage, out_hbm.at[pl.ds(row_base + 8 * p, 8)])
        return carry

    lax.fori_loop(0, NCHUNK // NRING, quad, 0)


def _mlp_body(m_ref, w1_ref, b1_ref, w2_ref, b2_ref, o_ref):
    m = m_ref[...]
    h = lax.dot_general(m, w1_ref[...], (((1,), (0,)), ((), ())),
                        preferred_element_type=jnp.float32)
    h = jnp.maximum(h + b1_ref[...], 0.0)
    o = lax.dot_general(h, w2_ref[...], (((1,), (0,)), ((), ())),
                        preferred_element_type=jnp.float32)
    o = o + b2_ref[...]
    n = jnp.sqrt(jnp.sum(o * o, axis=1, keepdims=True) + 1e-08)
    o_ref[...] = o / n


def _mlp(m, W1, b1, W2, b2):
    blk = 2048
    grid = (BATCH // blk,)
    return pl.pallas_call(
        _mlp_body,
        grid=grid,
        in_specs=[
            pl.BlockSpec((blk, EMBED_DIM), lambda i: (i, 0)),
            pl.BlockSpec((EMBED_DIM, HIDDEN_DIM), lambda i: (0, 0)),
            pl.BlockSpec((1, HIDDEN_DIM), lambda i: (0, 0)),
            pl.BlockSpec((HIDDEN_DIM, OUT_DIM), lambda i: (0, 0)),
            pl.BlockSpec((1, OUT_DIM), lambda i: (0, 0)),
        ],
        out_specs=pl.BlockSpec((blk, OUT_DIM), lambda i: (i, 0)),
        out_shape=jax.ShapeDtypeStruct((BATCH, OUT_DIM), jnp.float32),
    )(m, W1, b1.reshape(1, -1), W2, b2.reshape(1, -1))


def kernel(x, table, W1, b1, W2, b2):
    x = x.astype(jnp.int32)
    t2 = jnp.reshape(table, (VOCAB // 2, 2 * EMBED_DIM))
    t2lin = t2.reshape(VOCAB, EMBED_DIM)
    m = _pool(x.reshape(BATCH // CHUNK, ROWS), t2lin)
    return _mlp(m, W1, b1, W2, b2)


# speedup vs baseline: 1.9679x; 1.0020x over previous
import functools

import jax
import jax.numpy as jnp
from jax import lax
from jax.experimental import pallas as pl
from jax.experimental.pallas import tpu as pltpu
from jax.experimental.pallas import tpu_sc as plsc

EMBED_DIM = 64
HIDDEN_DIM = 128
OUT_DIM = 32
BATCH = 16384
HIST = 200
VOCAB = 1000000

NUM_WORKERS = 32
E_PER_W = BATCH // NUM_WORKERS
INV_H = 1.0 / HIST

CB = 256
NBLK_FULL = VOCAB // CB
TAIL_C0 = NBLK_FULL * CB
TAIL_W = VOCAB - TAIL_C0

CHUNK = 2
ROWS = CHUNK * HIST
NCHUNK = E_PER_W // CHUNK
XBLK = 32
NRING = 4

_mesh = plsc.VectorSubcoreMesh(core_axis_name="c", subcore_axis_name="s")


@functools.partial(
    pl.kernel,
    mesh=_mesh,
    out_type=jax.ShapeDtypeStruct((VOCAB // 2, 2 * EMBED_DIM), jnp.float32),
    scratch_types=[
        pltpu.VMEM((EMBED_DIM, CB), jnp.float32),
        pltpu.VMEM((EMBED_DIM, CB), jnp.float32),
        pltpu.VMEM((CB // 2, 2 * EMBED_DIM), jnp.float32),
        pltpu.VMEM((CB // 2, 2 * EMBED_DIM), jnp.float32),
        pltpu.VMEM((EMBED_DIM, TAIL_W), jnp.float32),
        pltpu.VMEM((TAIL_W // 2, 2 * EMBED_DIM), jnp.float32),
        pltpu.SemaphoreType.DMA,
        pltpu.SemaphoreType.DMA,
        pltpu.SemaphoreType.DMA,
        pltpu.SemaphoreType.DMA,
    ],
    compiler_params=pltpu.CompilerParams(needs_layout_passes=False,
                                         disable_bounds_checks=True),
)
def _relin(tabT_hbm, t2_hbm, vi0, vi1, vo0, vo1, vit, vot,
           si0, si1, so0, so1):
    wid = lax.axis_index("s") * 2 + lax.axis_index("c")
    vis = (vi0, vi1)
    vos = (vo0, vo1)
    sis = (si0, si1)
    sos = (so0, so1)
    row_iotas = [lax.iota(jnp.int32, 16) + 16 * q for q in range(4)]

    def blk(k):
        return wid + NUM_WORKERS * k

    def start_in(k, j):
        pltpu.make_async_copy(
            tabT_hbm.at[:, pl.ds(blk(k) * CB, CB)], vis[j], sis[j]).start()

    def wait_in(j):
        pltpu.make_async_copy(
            tabT_hbm.at[:, pl.ds(0, CB)], vis[j], sis[j]).wait()

    def start_out(k, j):
        pltpu.make_async_copy(
            vos[j], t2_hbm.at[pl.ds(blk(k) * (CB // 2), CB // 2)],
            sos[j]).start()

    def wait_out(j):
        pltpu.make_async_copy(
            vos[j], t2_hbm.at[pl.ds(0, CB // 2)], sos[j]).wait()

    def transpose(vi, vo, nrows):
        def body(j2, carry, vi=vi, vo=vo):
            for h in range(2):
                col = jnp.full((16,), 2 * j2 + h, jnp.int32)
                for q in range(4):
                    vals = plsc.load_gather(vi, [row_iotas[q], col])
                    vo[j2, pl.ds(64 * h + 16 * q, 16)] = vals
            return carry

        lax.fori_loop(0, nrows, body, 0, unroll=8)

    nblk = jnp.where(wid < NBLK_FULL - (NBLK_FULL // NUM_WORKERS) *
                     NUM_WORKERS, NBLK_FULL // NUM_WORKERS + 1,
                     NBLK_FULL // NUM_WORKERS)

    start_in(0, 0)
    start_in(1, 1)

    def pair(p, carry):
        k0 = 2 * p
        for j in range(2):
            k = k0 + j

            @pl.when(k < nblk)
            def _(k=k, j=j):
                wait_in(j)

                @pl.when(p > 0)
                def _():
                    wait_out(j)

                transpose(vis[j], vos[j], CB // 2)
                start_out(k, j)

                @pl.when(k + 2 < nblk)
                def _():
                    start_in(k + 2, j)

        return carry

    lax.fori_loop(0, (NBLK_FULL // NUM_WORKERS + 2) // 2, pair, 0)
    wait_out(0)
    wait_out(1)

    @pl.when(wid == 2)
    def _():
        pltpu.sync_copy(tabT_hbm.at[:, pl.ds(TAIL_C0, TAIL_W)], vit)
        transpose(vit, vot, TAIL_W // 2)
        pltpu.sync_copy(vot, t2_hbm.at[pl.ds(TAIL_C0 // 2, TAIL_W // 2)])


@functools.partial(
    pl.kernel,
    mesh=_mesh,
    out_type=jax.ShapeDtypeStruct((BATCH, EMBED_DIM), jnp.float32),
    scratch_types=[
        pltpu.VMEM((XBLK // CHUNK, ROWS), jnp.int32),
        pltpu.VMEM((XBLK // CHUNK, ROWS), jnp.int32),
        pltpu.VMEM((ROWS, EMBED_DIM), jnp.float32),
        pltpu.VMEM((ROWS, EMBED_DIM), jnp.float32),
        pltpu.VMEM((ROWS, EMBED_DIM), jnp.float32),
        pltpu.VMEM((ROWS, EMBED_DIM), jnp.float32),
        pltpu.VMEM((8, EMBED_DIM), jnp.float32),
        pltpu.SemaphoreType.DMA,
        pltpu.SemaphoreType.DMA,
        pltpu.SemaphoreType.DMA,
        pltpu.SemaphoreType.DMA,
    ],
    compiler_params=pltpu.CompilerParams(use_tc_tiling_on_sc=False),
)
def _pool(x_hbm, t2_hbm, out_hbm,
          xb0, xb1, rb0, rb1, rb2, rb3, stage, g0, g1, g2, g3):
    wid = lax.axis_index("s") * 2 + lax.axis_index("c")
    row_base = wid * E_PER_W
    zero = jnp.zeros((16,), jnp.float32)
    xbufs = (xb0, xb1)
    rbufs = (rb0, rb1, rb2, rb3)
    sems = (g0, g1, g2, g3)
    chunks_per_xblk = XBLK // CHUNK

    def stage_x(kx, xbuf):
        pltpu.sync_copy(
            x_hbm.at[pl.ds((row_base + kx * XBLK) // CHUNK,
                           XBLK // CHUNK), :], xbuf)

    def start_gather(c, j):
        kx = c // chunks_per_xblk
        lc = c - kx * chunks_per_xblk
        @pl.when(kx % 2 == 0)
        def _():
            pltpu.make_async_copy(
                t2_hbm.at[xbufs[0].at[lc, :]],
                rbufs[j], sems[j]).start()

        @pl.when(kx % 2 == 1)
        def _():
            pltpu.make_async_copy(
                t2_hbm.at[xbufs[1].at[lc, :]],
                rbufs[j], sems[j]).start()

    def accum(c, j):
        pltpu.make_async_copy(
            t2_hbm.at[xbufs[0].at[0, :]],
            rbufs[j], sems[j]).wait()
        rbuf = rbufs[j]
        for e in range(CHUNK):
            base_r = e * HIST

            def body(i, accs, base_r=base_r, rbuf=rbuf):
                a0, a1, a2, a3 = accs
                a0 = a0 + rbuf[base_r + i, pl.ds(0, 16)]
                a1 = a1 + rbuf[base_r + i, pl.ds(16, 16)]
                a2 = a2 + rbuf[base_r + i, pl.ds(32, 16)]
                a3 = a3 + rbuf[base_r + i, pl.ds(48, 16)]
                return (a0, a1, a2, a3)

            a0, a1, a2, a3 = lax.fori_loop(
                0, HIST, body, (zero, zero, zero, zero), unroll=8)
            srow = (2 * j + e) % 8
            stage[srow, pl.ds(0, 16)] = a0 * INV_H
            stage[srow, pl.ds(16, 16)] = a1 * INV_H
            stage[srow, pl.ds(32, 16)] = a2 * INV_H
            stage[srow, pl.ds(48, 16)] = a3 * INV_H

    stage_x(0, xb0)
    start_gather(0, 0)
    start_gather(1, 1)
    start_gather(2, 2)

    def quad(p, carry):
        for j in range(NRING):
            c = 4 * p + j

            @pl.when(c % chunks_per_xblk == 12)
            def _(c=c):
                kxn = c // chunks_per_xblk + 1

                @pl.when(kxn < E_PER_W // XBLK)
                def _():
                    @pl.when(kxn % 2 == 0)
                    def _():
                        stage_x(kxn, xbufs[0])

                    @pl.when(kxn % 2 == 1)
                    def _():
                        stage_x(kxn, xbufs[1])

            accum(c, j)

            @pl.when(c + 3 < NCHUNK)
            def _(c=c, j=j):
                start_gather(c + 3, (j + 3) % NRING)

        pltpu.sync_copy(
            stage, out_hbm.at[pl.ds(row_base + 8 * p, 8)])
        return carry

    lax.fori_loop(0, NCHUNK // NRING, quad, 0)


def _mlp_body(m_ref, w1_ref, b1_ref, w2_ref, b2_ref, o_ref):
    m = m_ref[...]
    h = lax.dot_general(m, w1_ref[...], (((1,), (0,)), ((), ())),
                        preferred_element_type=jnp.float32)
    h = jnp.maximum(h + b1_ref[...], 0.0)
    o = lax.dot_general(h, w2_ref[...], (((1,), (0,)), ((), ())),
                        preferred_element_type=jnp.float32)
    o = o + b2_ref[...]
    n = jnp.sqrt(jnp.sum(o * o, axis=1, keepdims=True) + 1e-08)
    o_ref[...] = o / n


def _mlp(m, W1, b1, W2, b2):
    blk = 2048
    grid = (BATCH // blk,)
    return pl.pallas_call(
        _mlp_body,
        grid=grid,
        in_specs=[
            pl.BlockSpec((blk, EMBED_DIM), lambda i: (i, 0)),
            pl.BlockSpec((EMBED_DIM, HIDDEN_DIM), lambda i: (0, 0)),
            pl.BlockSpec((1, HIDDEN_DIM), lambda i: (0, 0)),
            pl.BlockSpec((HIDDEN_DIM, OUT_DIM), lambda i: (0, 0)),
            pl.BlockSpec((1, OUT_DIM), lambda i: (0, 0)),
        ],
        out_specs=pl.BlockSpec((blk, OUT_DIM), lambda i: (i, 0)),
        out_shape=jax.ShapeDtypeStruct((BATCH, OUT_DIM), jnp.float32),
    )(m, W1, b1.reshape(1, -1), W2, b2.reshape(1, -1))


def kernel(x, table, W1, b1, W2, b2):
    x = x.astype(jnp.int32)
    t2 = jnp.reshape(table, (VOCAB // 2, 2 * EMBED_DIM))
    t2lin = t2.reshape(VOCAB, EMBED_DIM)
    m = _pool(x.reshape(BATCH // CHUNK, ROWS), t2lin)
    return _mlp(m, W1, b1, W2, b2)
